# baseline (speedup 1.0000x reference)
import jax
import jax.numpy as jnp
from jax import lax
from jax.experimental import pallas as pl
from jax.experimental.pallas import tpu as pltpu
from jax.experimental.pallas import tpu_sc as plsc


def _lane_take(v, idx):
    return v.at[idx].get(mode="promise_in_bounds")


def _transpose16(vs, iota):
    for s in (1, 2, 4, 8):
        perm = iota ^ s
        nv = []
        for i in range(16):
            pp = _lane_take(vs[i ^ s], perm)
            keep = (iota & s) == (i & s)
            nv.append(jnp.where(keep, vs[i], pp))
        vs = nv
    return vs


def _sc_lookup_body(rw_hbm, cw_hbm, out_hbm, rw_v, cw_v, slab_v, sem):
    wid = lax.axis_index("s") * 2 + lax.axis_index("c")

    pltpu.sync_copy(rw_hbm, rw_v)
    pltpu.sync_copy(cw_hbm, cw_v)

    iota16 = lax.iota(jnp.int32, 16)

    def build(tab_v, c0):
        for g in range(2):
            vs = [
                tab_v[pl.ds((g * 16 + i) * 256 + c0, 16)] for i in range(16)
            ]
            t = _transpose16(vs, iota16)
            for j in range(16):
                slab_v[pl.ds(j * 32 + g * 16, 16)] = t[j]

    @pl.when(wid < 16)
    def _build_x():
        build(rw_v, wid * 16)

    @pl.when(wid >= 16)
    def _build_y():
        build(cw_v, wid * 16 - 256)

    pltpu.async_copy(slab_v, out_hbm.at[pl.ds(wid * 512, 512)], sem).wait()


def _tc_broadcast_body(ps_ref, o_ref):
    ps = ps_ref[...]
    xs = ps[:256]
    ys = ps[256:]
    o_ref[0, :256] = jnp.broadcast_to(xs[:, None, :], (256, 32, 32))
    o_ref[0, 256:] = jnp.broadcast_to(ys[:, :, None], (256, 32, 32))


def kernel(uv_feat, row_weight, col_weight):
    b = uv_feat.shape[0]
    h, w = uv_feat.shape[-2], uv_feat.shape[-1]
    d = row_weight.shape[-1]
    assert (b, h, w, d) == (16, 32, 32, 256)

    mesh = plsc.VectorSubcoreMesh(core_axis_name="c", subcore_axis_name="s")
    sc_lookup = pl.kernel(
        _sc_lookup_body,
        mesh=mesh,
        out_type=jax.ShapeDtypeStruct((2 * d * 32,), jnp.float32),
        scratch_types=[
            pltpu.VMEM((w * d,), jnp.float32),
            pltpu.VMEM((h * d,), jnp.float32),
            pltpu.VMEM((512,), jnp.float32),
            pltpu.SemaphoreType.DMA,
        ],
    )
    ps = sc_lookup(
        row_weight[:w].reshape(w * d),
        col_weight[:h].reshape(h * d),
    ).reshape(2 * d, 32)

    return pl.pallas_call(
        _tc_broadcast_body,
        grid=(b,),
        in_specs=[pl.BlockSpec((2 * d, 32), lambda i: (0, 0))],
        out_specs=pl.BlockSpec((1, 2 * d, h, w), lambda i: (i, 0, 0, 0)),
        out_shape=jax.ShapeDtypeStruct((b, 2 * d, h, w), jnp.float32),
    )(ps)


# speedup vs baseline: 1.7978x; 1.7978x over previous
import jax
import jax.numpy as jnp
from jax import lax
from jax.experimental import pallas as pl
from jax.experimental.pallas import tpu as pltpu
from jax.experimental.pallas import tpu_sc as plsc


def _lane_take(v, idx):
    return v.at[idx].get(mode="promise_in_bounds")


def _transpose16(vs, iota):
    for s in (1, 2, 4, 8):
        perm = iota ^ s
        nv = []
        for i in range(16):
            pp = _lane_take(vs[i ^ s], perm)
            keep = (iota & s) == (i & s)
            nv.append(jnp.where(keep, vs[i], pp))
        vs = nv
    return vs


def _sc_lookup_body(rw_hbm, cw_hbm, out_hbm, rw_v, cw_v, slab_v, sem):
    wid = lax.axis_index("s") * 2 + lax.axis_index("c")

    pltpu.sync_copy(rw_hbm, rw_v)
    pltpu.sync_copy(cw_hbm, cw_v)

    iota16 = lax.iota(jnp.int32, 16)

    def build(tab_v, c0):
        for g in range(2):
            vs = [
                tab_v[pl.ds((g * 16 + i) * 256 + c0, 16)] for i in range(16)
            ]
            t = _transpose16(vs, iota16)
            for j in range(16):
                slab_v[pl.ds(j * 32 + g * 16, 16)] = t[j]

    @pl.when(wid < 16)
    def _build_x():
        build(rw_v, wid * 16)

    @pl.when(wid >= 16)
    def _build_y():
        build(cw_v, wid * 16 - 256)

    pltpu.async_copy(slab_v, out_hbm.at[pl.ds(wid * 512, 512)], sem).wait()


def _tc_broadcast_body(ps_ref, o_ref):
    ps = ps_ref[...]
    xs = ps[:256]
    ys = ps[256:]
    k = lax.broadcasted_iota(jnp.int32, (256, 1024), 1)
    x_tile = jnp.take_along_axis(xs, k % 32, axis=1)
    y_tile = jnp.take_along_axis(ys, k // 32, axis=1)
    o_ref[0, :256] = x_tile
    o_ref[0, 256:] = y_tile


def kernel(uv_feat, row_weight, col_weight):
    b = uv_feat.shape[0]
    h, w = uv_feat.shape[-2], uv_feat.shape[-1]
    d = row_weight.shape[-1]
    assert (b, h, w, d) == (16, 32, 32, 256)

    mesh = plsc.VectorSubcoreMesh(core_axis_name="c", subcore_axis_name="s")
    sc_lookup = pl.kernel(
        _sc_lookup_body,
        mesh=mesh,
        out_type=jax.ShapeDtypeStruct((2 * d * 32,), jnp.float32),
        scratch_types=[
            pltpu.VMEM((w * d,), jnp.float32),
            pltpu.VMEM((h * d,), jnp.float32),
            pltpu.VMEM((512,), jnp.float32),
            pltpu.SemaphoreType.DMA,
        ],
    )
    ps = sc_lookup(
        row_weight[:w].reshape(w * d),
        col_weight[:h].reshape(h * d),
    ).reshape(2 * d, 32)

    out = pl.pallas_call(
        _tc_broadcast_body,
        grid=(b,),
        in_specs=[pl.BlockSpec((2 * d, 32), lambda i: (0, 0))],
        out_specs=pl.BlockSpec((1, 2 * d, h * w), lambda i: (i, 0, 0)),
        out_shape=jax.ShapeDtypeStruct((b, 2 * d, h * w), jnp.float32),
    )(ps)
    return out.reshape(b, 2 * d, h, w)


# speedup vs baseline: 2.0988x; 1.1675x over previous
import jax
import jax.numpy as jnp
from jax import lax
from jax.experimental import pallas as pl
from jax.experimental.pallas import tpu as pltpu
from jax.experimental.pallas import tpu_sc as plsc


def _lane_take(v, idx):
    return v.at[idx].get(mode="promise_in_bounds")


def _transpose16(vs, iota):
    for s in (1, 2, 4, 8):
        perm = iota ^ s
        nv = []
        for i in range(16):
            pp = _lane_take(vs[i ^ s], perm)
            keep = (iota & s) == (i & s)
            nv.append(jnp.where(keep, vs[i], pp))
        vs = nv
    return vs


def _sc_lookup_body(rw_hbm, cw_hbm, out_hbm, rw_v, cw_v, slab_v, sem):
    wid = lax.axis_index("s") * 2 + lax.axis_index("c")

    pltpu.sync_copy(rw_hbm, rw_v)
    pltpu.sync_copy(cw_hbm, cw_v)

    iota16 = lax.iota(jnp.int32, 16)

    def build(tab_v, c0):
        for g in range(2):
            vs = [
                tab_v[pl.ds((g * 16 + i) * 256 + c0, 16)] for i in range(16)
            ]
            t = _transpose16(vs, iota16)
            for j in range(16):
                slab_v[pl.ds(j * 32 + g * 16, 16)] = t[j]

    @pl.when(wid < 16)
    def _build_x():
        build(rw_v, wid * 16)

    @pl.when(wid >= 16)
    def _build_y():
        build(cw_v, wid * 16 - 256)

    pltpu.async_copy(slab_v, out_hbm.at[pl.ds(wid * 512, 512)], sem).wait()


def _tc_broadcast_body(ps_ref, o_ref, tile_v, sem):
    ps = ps_ref[...]
    xs = ps[:256]
    ys = ps[256:]
    k = lax.broadcasted_iota(jnp.int32, (256, 1024), 1)
    tile_v[:256] = jnp.take_along_axis(xs, k % 32, axis=1)
    tile_v[256:] = jnp.take_along_axis(ys, k // 32, axis=1)
    copies = [
        pltpu.make_async_copy(tile_v, o_ref.at[b], sem) for b in range(16)
    ]
    for c in copies:
        c.start()
    for c in copies:
        c.wait()


def kernel(uv_feat, row_weight, col_weight):
    b = uv_feat.shape[0]
    h, w = uv_feat.shape[-2], uv_feat.shape[-1]
    d = row_weight.shape[-1]
    assert (b, h, w, d) == (16, 32, 32, 256)

    mesh = plsc.VectorSubcoreMesh(core_axis_name="c", subcore_axis_name="s")
    sc_lookup = pl.kernel(
        _sc_lookup_body,
        mesh=mesh,
        out_type=jax.ShapeDtypeStruct((2 * d * 32,), jnp.float32),
        scratch_types=[
            pltpu.VMEM((w * d,), jnp.float32),
            pltpu.VMEM((h * d,), jnp.float32),
            pltpu.VMEM((512,), jnp.float32),
            pltpu.SemaphoreType.DMA,
        ],
    )
    ps = sc_lookup(
        row_weight[:w].reshape(w * d),
        col_weight[:h].reshape(h * d),
    ).reshape(2 * d, 32)

    out = pl.pallas_call(
        _tc_broadcast_body,
        in_specs=[pl.BlockSpec((2 * d, 32), lambda: (0, 0))],
        out_specs=pl.BlockSpec(memory_space=pl.ANY),
        out_shape=jax.ShapeDtypeStruct((b, 2 * d, h * w), jnp.float32),
        scratch_shapes=[
            pltpu.VMEM((2 * d, h * w), jnp.float32),
            pltpu.SemaphoreType.DMA,
        ],
    )(ps)
    return out.reshape(b, 2 * d, h, w)


# speedup vs baseline: 2.1405x; 1.0199x over previous
import jax
import jax.numpy as jnp
from jax import lax
from jax.experimental import pallas as pl
from jax.experimental.pallas import tpu as pltpu
from jax.experimental.pallas import tpu_sc as plsc


def _lane_take(v, idx):
    return v.at[idx].get(mode="promise_in_bounds")


def _transpose16(vs, iota):
    for s in (1, 2, 4, 8):
        perm = iota ^ s
        nv = []
        for i in range(16):
            pp = _lane_take(vs[i ^ s], perm)
            keep = (iota & s) == (i & s)
            nv.append(jnp.where(keep, vs[i], pp))
        vs = nv
    return vs


def _sc_lookup_body(rw_hbm, cw_hbm, out_hbm, rw_v, cw_v, slab_v, sem):
    wid = lax.axis_index("s") * 2 + lax.axis_index("c")

    pltpu.sync_copy(rw_hbm.at[pl.ds(0, 32)], rw_v)
    pltpu.sync_copy(cw_hbm.at[pl.ds(0, 32)], cw_v)

    iota16 = lax.iota(jnp.int32, 16)

    def build(tab_v, c0):
        for g in range(2):
            vs = [tab_v[g * 16 + i, pl.ds(c0, 16)] for i in range(16)]
            t = _transpose16(vs, iota16)
            for j in range(16):
                slab_v[j, pl.ds(g * 16, 16)] = t[j]

    @pl.when(wid < 16)
    def _build_x():
        build(rw_v, wid * 16)

    @pl.when(wid >= 16)
    def _build_y():
        build(cw_v, wid * 16 - 256)

    pltpu.async_copy(slab_v, out_hbm.at[pl.ds(wid * 16, 16)], sem).wait()


def _tc_broadcast_body(ps_ref, o_ref, tile_v, sem):
    ps = ps_ref[...]
    xs = ps[:256]
    ys = ps[256:]
    k = lax.broadcasted_iota(jnp.int32, (256, 1024), 1)
    tile_v[:256] = jnp.take_along_axis(xs, k % 32, axis=1)
    tile_v[256:] = jnp.take_along_axis(ys, k // 32, axis=1)
    copies = [
        pltpu.make_async_copy(tile_v, o_ref.at[b], sem) for b in range(16)
    ]
    for c in copies:
        c.start()
    for c in copies:
        c.wait()


def kernel(uv_feat, row_weight, col_weight):
    b = uv_feat.shape[0]
    h, w = uv_feat.shape[-2], uv_feat.shape[-1]
    d = row_weight.shape[-1]
    assert (b, h, w, d) == (16, 32, 32, 256)

    mesh = plsc.VectorSubcoreMesh(core_axis_name="c", subcore_axis_name="s")
    sc_lookup = pl.kernel(
        _sc_lookup_body,
        mesh=mesh,
        out_type=jax.ShapeDtypeStruct((2 * d, 32), jnp.float32),
        scratch_types=[
            pltpu.VMEM((w, d), jnp.float32),
            pltpu.VMEM((h, d), jnp.float32),
            pltpu.VMEM((16, 32), jnp.float32),
            pltpu.SemaphoreType.DMA,
        ],
    )
    ps = sc_lookup(row_weight, col_weight)

    out = pl.pallas_call(
        _tc_broadcast_body,
        in_specs=[pl.BlockSpec((2 * d, 32), lambda: (0, 0))],
        out_specs=pl.BlockSpec(memory_space=pl.ANY),
        out_shape=jax.ShapeDtypeStruct((b, 2 * d, h * w), jnp.float32),
        scratch_shapes=[
            pltpu.VMEM((2 * d, h * w), jnp.float32),
            pltpu.SemaphoreType.DMA,
        ],
    )(ps)
    return out.reshape(b, 2 * d, h, w)
